# baseline (speedup 1.0000x reference)
import functools

import jax
import jax.numpy as jnp
import numpy as np
from jax import lax
from jax.experimental import pallas as pl
from jax.experimental.pallas import tpu as pltpu
from jax.experimental.pallas import tpu_sc as plsc

DIM = 1024
GRID = 16
XMIN = -3.5
XMAX = 3.5
N_ROWS = 32768

LANES = 16
NUM_WORKERS = 32
TOTAL = N_ROWS * DIM
PER_W = TOTAL // NUM_WORKERS
CHUNK = 16 * DIM
NCHUNK = PER_W // CHUNK
VPC = CHUNK // LANES

_H = np.float32((XMAX - XMIN) / (GRID - 1))
_INV_H = np.float32(1.0) / _H
_C0 = np.float32(-XMIN) * _INV_H


def _spline_body(a_hbm, b_hbm, x_hbm, o_hbm, tab_a, tab_b, xbuf, obuf,
                 si0, si1, so0, so1):
    wid = lax.axis_index("s") * 2 + lax.axis_index("c")
    base = wid * PER_W

    pltpu.sync_copy(a_hbm, tab_a)
    pltpu.sync_copy(b_hbm, tab_b)

    lane = lax.iota(jnp.int32, LANES)
    sin = (si0, si1)
    sout = (so0, so1)

    def in_src(g):
        return x_hbm.at[pl.ds(base + g * CHUNK, CHUNK)]

    def out_dst(g):
        return o_hbm.at[pl.ds(base + g * CHUNK, CHUNK)]

    pltpu.async_copy(in_src(0), xbuf.at[0], si0)
    pltpu.async_copy(in_src(1), xbuf.at[1], si1)

    def step(i, _):
        for b in range(2):
            g = i * 2 + b
            pltpu.make_async_copy(in_src(g), xbuf.at[b], sin[b]).wait()

            @pl.when(i >= 1)
            def _():
                pltpu.make_async_copy(obuf.at[b], out_dst(g - 2), sout[b]).wait()

            @plsc.parallel_loop(0, DIM // LANES, unroll=2)
            def _(c):
                colvec = c * LANES + lane
                cbase = c * LANES
                for r in range(CHUNK // DIM):
                    o = r * DIM + cbase
                    xv = xbuf[b, pl.ds(o, LANES)]
                    xc = jnp.minimum(jnp.maximum(xv, XMIN), XMAX)
                    u = xc * _INV_H + _C0
                    i0 = jnp.minimum(u.astype(jnp.int32), GRID - 2)
                    idx = (i0 << 10) + colvec
                    av = plsc.load_gather(tab_a, [idx])
                    bv = plsc.load_gather(tab_b, [idx])
                    obuf[b, pl.ds(o, LANES)] = bv * xc + av

            pltpu.async_copy(obuf.at[b], out_dst(g), sout[b])

            @pl.when(i < NCHUNK // 2 - 1)
            def _():
                pltpu.async_copy(in_src(g + 2), xbuf.at[b], sin[b])

        return 0

    lax.fori_loop(0, NCHUNK // 2, step, 0)

    pltpu.make_async_copy(obuf.at[0], out_dst(NCHUNK - 2), so0).wait()
    pltpu.make_async_copy(obuf.at[1], out_dst(NCHUNK - 1), so1).wait()


@jax.jit
def _spline_sc(a16, b16, x_flat):
    mesh = plsc.VectorSubcoreMesh(core_axis_name="c", subcore_axis_name="s")
    return pl.kernel(
        _spline_body,
        mesh=mesh,
        compiler_params=pltpu.CompilerParams(needs_layout_passes=False),
        out_type=jax.ShapeDtypeStruct((TOTAL,), jnp.float32),
        scratch_types=[
            pltpu.VMEM((GRID * DIM,), jnp.float32),
            pltpu.VMEM((GRID * DIM,), jnp.float32),
            pltpu.VMEM((2, CHUNK), jnp.float32),
            pltpu.VMEM((2, CHUNK), jnp.float32),
            pltpu.SemaphoreType.DMA,
            pltpu.SemaphoreType.DMA,
            pltpu.SemaphoreType.DMA,
            pltpu.SemaphoreType.DMA,
        ],
    )(a16, b16, x_flat)


def kernel(x, coeff, base_scale, base_bias):
    grid = jnp.linspace(XMIN, XMAX, GRID).astype(jnp.float32)
    y0 = coeff[:, :-1]
    y1 = coeff[:, 1:]
    s = (y1 - y0) / (grid[1:] - grid[:-1] + 1e-8)
    b_t = base_scale[:, None] + s
    a_t = base_bias[:, None] + y0 - s * grid[:-1]
    a16 = jnp.concatenate([a_t, a_t[:, -1:]], axis=1).T.reshape(-1)
    b16 = jnp.concatenate([b_t, b_t[:, -1:]], axis=1).T.reshape(-1)

    out_flat = _spline_sc(a16, b16, x.reshape(-1))
    return out_flat.reshape(N_ROWS, DIM)


# speedup vs baseline: 1.0963x; 1.0963x over previous
import jax
import jax.numpy as jnp
import numpy as np
from jax import lax
from jax.experimental import pallas as pl
from jax.experimental.pallas import tpu as pltpu
from jax.experimental.pallas import tpu_sc as plsc

DIM = 1024
GRID = 16
XMIN = -3.5
XMAX = 3.5
N_ROWS = 32768

LANES = 16
NUM_WORKERS = 32
TOTAL = N_ROWS * DIM
PER_W = TOTAL // NUM_WORKERS
CHUNK = 16 * DIM
NCHUNK = PER_W // CHUNK
VPC = CHUNK // LANES
CGRP = DIM // LANES

_H = np.float32((XMAX - XMIN) / (GRID - 1))
_INV_H = np.float32(1.0) / _H
_C0 = np.float32(-XMIN) * _INV_H


def _spline_body(a_hbm, b_hbm, x_hbm, o_hbm, tab_a, tab_b, xbuf, obuf,
                 si0, si1, so0, so1):
    wid = lax.axis_index("s") * 2 + lax.axis_index("c")
    base = wid * PER_W

    pltpu.sync_copy(a_hbm, tab_a)
    pltpu.sync_copy(b_hbm, tab_b)

    lane16 = lax.iota(jnp.int32, LANES) * GRID
    sin = (si0, si1)
    sout = (so0, so1)

    def in_src(g):
        return x_hbm.at[pl.ds(base + g * CHUNK, CHUNK)]

    def out_dst(g):
        return o_hbm.at[pl.ds(base + g * CHUNK, CHUNK)]

    pltpu.async_copy(in_src(0), xbuf.at[0], si0)
    pltpu.async_copy(in_src(1), xbuf.at[1], si1)

    def step(i, _):
        for b in range(2):
            g = i * 2 + b
            pltpu.make_async_copy(in_src(g), xbuf.at[b], sin[b]).wait()

            @pl.when(i >= 1)
            def _():
                pltpu.make_async_copy(obuf.at[b], out_dst(g - 2), sout[b]).wait()

            @plsc.parallel_loop(0, VPC, unroll=8)
            def _(k):
                o = k * LANES
                cb16 = (k & (CGRP - 1)) * (LANES * GRID)
                xv = xbuf[b, pl.ds(o, LANES)]
                u0 = xv * _INV_H + _C0
                u = jnp.minimum(jnp.maximum(u0, 0.0), np.float32(GRID - 1))
                idx = u.astype(jnp.int32) + lane16
                av = plsc.load_gather(tab_a.at[pl.ds(cb16, LANES * GRID)], [idx])
                bv = plsc.load_gather(tab_b.at[pl.ds(cb16, LANES * GRID)], [idx])
                obuf[b, pl.ds(o, LANES)] = bv * u + av

            pltpu.async_copy(obuf.at[b], out_dst(g), sout[b])

            @pl.when(i < NCHUNK // 2 - 1)
            def _():
                pltpu.async_copy(in_src(g + 2), xbuf.at[b], sin[b])

        return 0

    lax.fori_loop(0, NCHUNK // 2, step, 0)

    pltpu.make_async_copy(obuf.at[0], out_dst(NCHUNK - 2), so0).wait()
    pltpu.make_async_copy(obuf.at[1], out_dst(NCHUNK - 1), so1).wait()


@jax.jit
def _spline_sc(a16, b16, x_flat):
    mesh = plsc.VectorSubcoreMesh(core_axis_name="c", subcore_axis_name="s")
    return pl.kernel(
        _spline_body,
        mesh=mesh,
        compiler_params=pltpu.CompilerParams(needs_layout_passes=False),
        out_type=jax.ShapeDtypeStruct((TOTAL,), jnp.float32),
        scratch_types=[
            pltpu.VMEM((DIM * GRID,), jnp.float32),
            pltpu.VMEM((DIM * GRID,), jnp.float32),
            pltpu.VMEM((2, CHUNK), jnp.float32),
            pltpu.VMEM((2, CHUNK), jnp.float32),
            pltpu.SemaphoreType.DMA,
            pltpu.SemaphoreType.DMA,
            pltpu.SemaphoreType.DMA,
            pltpu.SemaphoreType.DMA,
        ],
    )(a16, b16, x_flat)


def kernel(x, coeff, base_scale, base_bias):
    grid = jnp.linspace(XMIN, XMAX, GRID).astype(jnp.float32)
    y0 = coeff[:, :-1]
    y1 = coeff[:, 1:]
    s = (y1 - y0) / (grid[1:] - grid[:-1] + 1e-8)
    b_x = base_scale[:, None] + s
    a_x = base_bias[:, None] + y0 - s * grid[:-1]
    b_u = b_x * _H
    a_u = a_x + b_x * np.float32(XMIN)
    a16 = jnp.concatenate([a_u, a_u[:, -1:]], axis=1).reshape(-1)
    b16 = jnp.concatenate([b_u, b_u[:, -1:]], axis=1).reshape(-1)

    out_flat = _spline_sc(a16, b16, x.reshape(-1))
    return out_flat.reshape(N_ROWS, DIM)


# speedup vs baseline: 1.2168x; 1.1099x over previous
import jax
import jax.numpy as jnp
import numpy as np
from jax import lax
from jax.experimental import pallas as pl
from jax.experimental.pallas import tpu as pltpu
from jax.experimental.pallas import tpu_sc as plsc

DIM = 1024
GRID = 16
XMIN = -3.5
XMAX = 3.5
N_ROWS = 32768

LANES = 16
NUM_WORKERS = 32
TOTAL = N_ROWS * DIM
PER_W = TOTAL // NUM_WORKERS
CHUNK = 16 * DIM
NCHUNK = PER_W // CHUNK
VPC = CHUNK // LANES
CGRP = DIM // LANES

TSLICE = (GRID - 1) * DIM + LANES

_H = np.float32((XMAX - XMIN) / (GRID - 1))
_INV_H = np.float32(1.0) / _H
_C0 = np.float32(-XMIN) * _INV_H


def _spline_body(a_hbm, b_hbm, x_hbm, o_hbm, tab_a, tab_b, xbuf, obuf,
                 si0, si1, so0, so1):
    wid = lax.axis_index("s") * 2 + lax.axis_index("c")
    base = wid * PER_W

    pltpu.sync_copy(a_hbm, tab_a)
    pltpu.sync_copy(b_hbm, tab_b)

    lane = lax.iota(jnp.int32, LANES)
    sin = (si0, si1)
    sout = (so0, so1)

    def in_src(g):
        return x_hbm.at[pl.ds(base + g * CHUNK, CHUNK)]

    def out_dst(g):
        return o_hbm.at[pl.ds(base + g * CHUNK, CHUNK)]

    pltpu.async_copy(in_src(0), xbuf.at[0], si0)
    pltpu.async_copy(in_src(1), xbuf.at[1], si1)

    def step(i, _):
        for b in range(2):
            g = i * 2 + b
            pltpu.make_async_copy(in_src(g), xbuf.at[b], sin[b]).wait()

            @pl.when(i >= 1)
            def _():
                pltpu.make_async_copy(obuf.at[b], out_dst(g - 2), sout[b]).wait()

            @plsc.parallel_loop(0, VPC, unroll=8)
            def _(k):
                o = k * LANES
                cb = (k & (CGRP - 1)) * LANES
                xv = xbuf[b, pl.ds(o, LANES)]
                u0 = xv * _INV_H + _C0
                u = jnp.minimum(jnp.maximum(u0, 0.0), np.float32(GRID - 1))
                idx = (u.astype(jnp.int32) << 10) + lane
                av = plsc.load_gather(tab_a.at[pl.ds(cb, TSLICE)], [idx])
                bv = plsc.load_gather(tab_b.at[pl.ds(cb, TSLICE)], [idx])
                obuf[b, pl.ds(o, LANES)] = bv * u + av

            pltpu.async_copy(obuf.at[b], out_dst(g), sout[b])

            @pl.when(i < NCHUNK // 2 - 1)
            def _():
                pltpu.async_copy(in_src(g + 2), xbuf.at[b], sin[b])

        return 0

    lax.fori_loop(0, NCHUNK // 2, step, 0)

    pltpu.make_async_copy(obuf.at[0], out_dst(NCHUNK - 2), so0).wait()
    pltpu.make_async_copy(obuf.at[1], out_dst(NCHUNK - 1), so1).wait()


@jax.jit
def _spline_sc(a16, b16, x_flat):
    mesh = plsc.VectorSubcoreMesh(core_axis_name="c", subcore_axis_name="s")
    return pl.kernel(
        _spline_body,
        mesh=mesh,
        compiler_params=pltpu.CompilerParams(needs_layout_passes=False),
        out_type=jax.ShapeDtypeStruct((TOTAL,), jnp.float32),
        scratch_types=[
            pltpu.VMEM((DIM * GRID,), jnp.float32),
            pltpu.VMEM((DIM * GRID,), jnp.float32),
            pltpu.VMEM((2, CHUNK), jnp.float32),
            pltpu.VMEM((2, CHUNK), jnp.float32),
            pltpu.SemaphoreType.DMA,
            pltpu.SemaphoreType.DMA,
            pltpu.SemaphoreType.DMA,
            pltpu.SemaphoreType.DMA,
        ],
    )(a16, b16, x_flat)


def kernel(x, coeff, base_scale, base_bias):
    grid = jnp.linspace(XMIN, XMAX, GRID).astype(jnp.float32)
    y0 = coeff[:, :-1]
    y1 = coeff[:, 1:]
    s = (y1 - y0) / (grid[1:] - grid[:-1] + 1e-8)
    b_x = base_scale[:, None] + s
    a_x = base_bias[:, None] + y0 - s * grid[:-1]
    b_u = b_x * _H
    a_u = a_x + b_x * np.float32(XMIN)
    a16 = jnp.concatenate([a_u, a_u[:, -1:]], axis=1).T.reshape(-1)
    b16 = jnp.concatenate([b_u, b_u[:, -1:]], axis=1).T.reshape(-1)

    out_flat = _spline_sc(a16, b16, x.reshape(-1))
    return out_flat.reshape(N_ROWS, DIM)
